# baseline (speedup 1.0000x reference)
import functools

import jax
import jax.numpy as jnp
from jax.experimental import pallas as pl
from jax.experimental.pallas import tpu as pltpu


def _act(th):
    t2 = th * th
    return (th + t2) / (1.0 + t2)


def _lattice_kernel(L, G, x1_ref, x2_ref, x3_ref, x4_ref,
                    wf_ref, wb_ref, bf_ref, bb_ref, out_ref, hf_scr, hb_scr):
    T = x1_ref.shape[1]
    H = wf_ref.shape[1]
    wf = 0.5 * wf_ref[...]
    wb = 0.5 * wb_ref[...]
    bf = 0.5 * bf_ref[...]
    bb = 0.5 * bb_ref[...]
    p = jax.lax.broadcasted_iota(jnp.int32, (L, 1), 0)
    zero = jnp.zeros((), jnp.float32)
    cnt_f = jnp.minimum(p + 1, 4).astype(jnp.float32)
    cnt_b = jnp.minimum(L - p, 4).astype(jnp.float32)

    for g in range(G):
        for q, x_ref in enumerate((x1_ref, x2_ref, x3_ref, x4_ref)):
            x = x_ref[g]
            hf_scr[q * T:(q + 1) * T, :] = _act(jnp.tanh(
                jnp.dot(x, wf, preferred_element_type=jnp.float32) + bf))
            hb_scr[q * T:(q + 1) * T, :] = _act(jnp.tanh(
                jnp.dot(x, wb, preferred_element_type=jnp.float32) + bb))

        hf = hf_scr[...]
        f1 = hf[0:L]
        f2 = jnp.where(p >= 1, hf[L - 1:2 * L - 1], zero)
        f3 = jnp.where(p >= 2, hf[2 * L - 3:3 * L - 3], zero)
        f4 = jnp.where(p >= 3, hf[3 * L - 6:4 * L - 6], zero)
        out_ref[g, :, :H] = (f1 + f2 + f3 + f4) / cnt_f

        hb = hb_scr[...]
        b1 = hb[0:L]
        b2 = jnp.where(p <= L - 2, hb[L:2 * L], zero)
        b3 = jnp.where(p <= L - 3, hb[2 * L - 1:3 * L - 1], zero)
        b4 = jnp.where(p <= L - 4, hb[3 * L - 3:4 * L - 3], zero)
        out_ref[g, :, H:] = (b1 + b2 + b3 + b4) / cnt_b


def kernel(edge_input, edge_begin, edge_end, W_ih_f, W_hh_f, b_f, W_ih_b, W_hh_b, b_b):
    del edge_begin, edge_end, W_hh_f, W_hh_b
    B, E, D = edge_input.shape
    H = W_ih_f.shape[1] // 4
    L = (E + 6) // 4
    T = (E + 31) // 32 * 8
    G = 4

    xspec = lambda q: pl.BlockSpec((G, T, D), lambda i, q=q: (i, q, 0))
    out = pl.pallas_call(
        functools.partial(_lattice_kernel, L, G),
        grid=(B // G,),
        in_specs=[
            xspec(0), xspec(1), xspec(2), xspec(3),
            pl.BlockSpec((D, H), lambda i: (0, 3)),
            pl.BlockSpec((D, H), lambda i: (0, 3)),
            pl.BlockSpec((1, H), lambda i: (0, 3)),
            pl.BlockSpec((1, H), lambda i: (0, 3)),
        ],
        out_specs=pl.BlockSpec((G, L, 2 * H), lambda i: (i, 0, 0)),
        out_shape=jax.ShapeDtypeStruct((B, L, 2 * H), jnp.float32),
        scratch_shapes=[
            pltpu.VMEM((4 * T, H), jnp.float32),
            pltpu.VMEM((4 * T, H), jnp.float32),
        ],
    )(edge_input, edge_input, edge_input, edge_input,
      W_ih_f, W_ih_b, b_f[None, :], b_b[None, :])
    return out


# speedup vs baseline: 1.0248x; 1.0248x over previous
import functools

import jax
import jax.numpy as jnp
from jax.experimental import pallas as pl
from jax.experimental.pallas import tpu as pltpu


def _act(th):
    t2 = th * th
    return (th + t2) / (1.0 + t2)


def _lattice_kernel(L, G, x1_ref, x2_ref, x3_ref, x4_ref,
                    wf_ref, wb_ref, bf_ref, bb_ref, out_ref, hf_scr, hb_scr):
    T = x1_ref.shape[1]
    H = wf_ref.shape[1]
    wf = 0.5 * wf_ref[...]
    wb = 0.5 * wb_ref[...]
    bf = 0.5 * bf_ref[...]
    bb = 0.5 * bb_ref[...]
    p = jax.lax.broadcasted_iota(jnp.int32, (L, 1), 0).astype(jnp.float32)
    rcp_f = 1.0 / jnp.minimum(p + 1.0, 4.0)
    rcp_b = 1.0 / jnp.minimum(L - p, 4.0)
    q = jax.lax.broadcasted_iota(jnp.int32, (8, 1), 0)
    zero = jnp.zeros((), jnp.float32)

    for g in range(G):
        for j, x_ref in enumerate((x1_ref, x2_ref, x3_ref, x4_ref)):
            x = x_ref[g]
            hf_scr[j * T:(j + 1) * T, :] = _act(jnp.tanh(
                jnp.dot(x, wf, preferred_element_type=jnp.float32) + bf))
            hb_scr[j * T:(j + 1) * T, :] = _act(jnp.tanh(
                jnp.dot(x, wb, preferred_element_type=jnp.float32) + bb))

        hf = hf_scr[...]
        uf = hf[0:L] + hf[L - 1:2 * L - 1] + hf[2 * L - 3:3 * L - 3] \
            + hf[3 * L - 6:4 * L - 6]
        out_ref[g, :, :H] = uf * rcp_f
        h2 = jnp.where(q >= 1, hf[L - 1:L + 7], zero)
        h3 = jnp.where(q >= 2, hf[2 * L - 3:2 * L + 5], zero)
        h4 = jnp.where(q >= 3, hf[3 * L - 6:3 * L + 2], zero)
        out_ref[g, 0:8, :H] = (hf[0:8] + h2 + h3 + h4) * rcp_f[0:8]

        hb = hb_scr[...]
        ub = hb[0:L] + hb[L:2 * L] + hb[2 * L - 1:3 * L - 1] \
            + hb[3 * L - 3:4 * L - 3]
        out_ref[g, :, H:] = ub * rcp_b
        t2_ = jnp.where(q <= 6, hb[2 * L - 8:2 * L], zero)
        t3_ = jnp.where(q <= 5, hb[3 * L - 9:3 * L - 1], zero)
        t4_ = jnp.where(q <= 4, hb[4 * L - 11:4 * L - 3], zero)
        out_ref[g, L - 8:L, H:] = (hb[L - 8:L] + t2_ + t3_ + t4_) * rcp_b[L - 8:L]


def kernel(edge_input, edge_begin, edge_end, W_ih_f, W_hh_f, b_f, W_ih_b, W_hh_b, b_b):
    del edge_begin, edge_end, W_hh_f, W_hh_b
    B, E, D = edge_input.shape
    H = W_ih_f.shape[1] // 4
    L = (E + 6) // 4
    T = (E + 31) // 32 * 8
    G = 2

    xspec = lambda j: pl.BlockSpec((G, T, D), lambda i, j=j: (i, j, 0))
    out = pl.pallas_call(
        functools.partial(_lattice_kernel, L, G),
        grid=(B // G,),
        in_specs=[
            xspec(0), xspec(1), xspec(2), xspec(3),
            pl.BlockSpec((D, H), lambda i: (0, 3)),
            pl.BlockSpec((D, H), lambda i: (0, 3)),
            pl.BlockSpec((1, H), lambda i: (0, 3)),
            pl.BlockSpec((1, H), lambda i: (0, 3)),
        ],
        out_specs=pl.BlockSpec((G, L, 2 * H), lambda i: (i, 0, 0)),
        out_shape=jax.ShapeDtypeStruct((B, L, 2 * H), jnp.float32),
        scratch_shapes=[
            pltpu.VMEM((4 * T, H), jnp.float32),
            pltpu.VMEM((4 * T, H), jnp.float32),
        ],
    )(edge_input, edge_input, edge_input, edge_input,
      W_ih_f, W_ih_b, b_f[None, :], b_b[None, :])
    return out
